# baseline (speedup 1.0000x reference)
import jax
import jax.numpy as jnp
from jax.experimental import pallas as pl
from jax.experimental.pallas import tpu as pltpu

SUF_CHUNK = 200
CTX_REP = 200


def _body(pre_hbm, ctx_hbm, suf_hbm, out_hbm, rep_v, sem_st, sem_pre, sem_ctx, sem_suf):
    n_cls, d = pre_hbm.shape
    cd = ctx_hbm.shape[1]
    sd = suf_hbm.shape[1]

    cp = pltpu.make_async_copy(ctx_hbm, rep_v.at[0:1], sem_st)
    cp.start()
    cp.wait()
    rep_v[...] = jnp.broadcast_to(rep_v[0:1], (CTX_REP, cd))

    pre_cp = pltpu.make_async_copy(pre_hbm, out_hbm.at[:, 0:d], sem_pre)
    pre_cp.start()

    suf_cps = []
    for k in range(n_cls // SUF_CHUNK):
        cp_k = pltpu.make_async_copy(
            suf_hbm.at[pl.ds(k * SUF_CHUNK, SUF_CHUNK)],
            out_hbm.at[pl.ds(k * SUF_CHUNK, SUF_CHUNK), pl.ds(d + cd, sd)],
            sem_suf,
        )
        cp_k.start()
        suf_cps.append(cp_k)

    ctx_cps = []
    for k in range(n_cls // CTX_REP):
        cp_k = pltpu.make_async_copy(
            rep_v,
            out_hbm.at[pl.ds(k * CTX_REP, CTX_REP), pl.ds(d, cd)],
            sem_ctx,
        )
        cp_k.start()
        ctx_cps.append(cp_k)

    pre_cp.wait()
    for cp_k in suf_cps:
        cp_k.wait()
    for cp_k in ctx_cps:
        cp_k.wait()


def kernel(ctx, token_prefix, token_suffix):
    n_cls, _, d = token_prefix.shape
    n_ctx = ctx.shape[0]
    s = token_suffix.shape[1]
    seq = 1 + n_ctx + s

    pre2 = token_prefix.reshape(n_cls, d)
    suf2 = token_suffix.reshape(n_cls, s * d)
    ctx2 = ctx.reshape(1, n_ctx * d)

    out = pl.pallas_call(
        _body,
        in_specs=[
            pl.BlockSpec(memory_space=pl.ANY),
            pl.BlockSpec(memory_space=pl.ANY),
            pl.BlockSpec(memory_space=pl.ANY),
        ],
        out_specs=pl.BlockSpec(memory_space=pl.ANY),
        out_shape=jax.ShapeDtypeStruct((n_cls, seq * d), jnp.float32),
        scratch_shapes=[
            pltpu.VMEM((CTX_REP, n_ctx * d), jnp.float32),
            pltpu.SemaphoreType.DMA,
            pltpu.SemaphoreType.DMA,
            pltpu.SemaphoreType.DMA,
            pltpu.SemaphoreType.DMA,
        ],
    )(pre2, ctx2, suf2)
    return out.reshape(n_cls, seq, d)


# speedup vs baseline: 10.4779x; 10.4779x over previous
import jax
import jax.numpy as jnp
from jax.experimental import pallas as pl


def _body(pre_ref, ctx_ref, suf_ref, out_ref):
    d = pre_ref.shape[1]
    cd = ctx_ref.shape[1]
    b = out_ref.shape[0]
    out_ref[:, 0:d] = pre_ref[...]
    out_ref[:, d:d + cd] = jnp.broadcast_to(ctx_ref[...], (b, cd))
    out_ref[:, d + cd:] = suf_ref[...]


def kernel(ctx, token_prefix, token_suffix):
    n_cls, _, d = token_prefix.shape
    n_ctx = ctx.shape[0]
    s = token_suffix.shape[1]
    seq = 1 + n_ctx + s

    pre2 = token_prefix.reshape(n_cls, d)
    suf2 = token_suffix.reshape(n_cls, s * d)
    ctx2 = ctx.reshape(1, n_ctx * d)

    B = 40
    out = pl.pallas_call(
        _body,
        grid=(n_cls // B,),
        in_specs=[
            pl.BlockSpec((B, d), lambda i: (i, 0)),
            pl.BlockSpec((1, n_ctx * d), lambda i: (0, 0)),
            pl.BlockSpec((B, s * d), lambda i: (i, 0)),
        ],
        out_specs=pl.BlockSpec((B, seq * d), lambda i: (i, 0)),
        out_shape=jax.ShapeDtypeStruct((n_cls, seq * d), jnp.float32),
    )(pre2, ctx2, suf2)
    return out.reshape(n_cls, seq, d)


# speedup vs baseline: 24.3740x; 2.3262x over previous
import jax
import jax.numpy as jnp
from jax.experimental import pallas as pl


def _body(pre_ref, ctx_ref, suf_ref, out_ref):
    b, _, d = pre_ref.shape
    n_ctx = ctx_ref.shape[0]
    s = suf_ref.shape[1]
    out_ref[:, 0:1, :] = pre_ref[...]
    out_ref[:, 1:1 + n_ctx, :] = jnp.broadcast_to(ctx_ref[...][None], (b, n_ctx, d))
    out_ref[:, 1 + n_ctx:, :] = suf_ref[...]


def kernel(ctx, token_prefix, token_suffix):
    n_cls, _, d = token_prefix.shape
    n_ctx = ctx.shape[0]
    s = token_suffix.shape[1]
    seq = 1 + n_ctx + s

    B = 40
    return pl.pallas_call(
        _body,
        grid=(n_cls // B,),
        in_specs=[
            pl.BlockSpec((B, 1, d), lambda i: (i, 0, 0)),
            pl.BlockSpec((n_ctx, d), lambda i: (0, 0)),
            pl.BlockSpec((B, s, d), lambda i: (i, 0, 0)),
        ],
        out_specs=pl.BlockSpec((B, seq, d), lambda i: (i, 0, 0)),
        out_shape=jax.ShapeDtypeStruct((n_cls, seq, d), jnp.float32),
    )(token_prefix, ctx, token_suffix)
